# Initial kernel scaffold; baseline (speedup 1.0000x reference)
import jax
import jax.numpy as jnp
from jax.experimental import pallas as pl


def kernel(coords, pe):
    raise NotImplementedError("write your pallas kernel here")


# speedup vs baseline: 6.8771x; 6.8771x over previous
import functools

import jax
import jax.numpy as jnp
from jax import lax
from jax.experimental import pallas as pl
from jax.experimental.pallas import tpu as pltpu
from jax.experimental.pallas import tpu_sc as plsc

_NC = 2
_NS = 16
_NW = _NC * _NS
_L = 16


@functools.lru_cache(maxsize=None)
def _make_sc_kernel(N, D, C):
    assert N % _NW == 0
    rows_per_tile = N // _NW
    assert rows_per_tile % C == 0
    nchunks = rows_per_tile // C
    assert D % _L == 0

    mesh = plsc.VectorSubcoreMesh(core_axis_name="c", subcore_axis_name="s")

    @functools.partial(
        pl.kernel,
        out_type=jax.ShapeDtypeStruct((N, D), jnp.float32),
        mesh=mesh,
        scratch_types=[
            pltpu.VMEM((C,), jnp.int32),
            pltpu.VMEM((C,), jnp.int32),
            pltpu.VMEM((C, D), jnp.float32),
            pltpu.VMEM((C, D), jnp.float32),
            pltpu.VMEM((C, D), jnp.float32),
            pltpu.SemaphoreType.DMA,
            pltpu.SemaphoreType.DMA,
        ],
    )
    def k(xs_hbm, ys_hbm, pe_hbm, out_hbm, idxx, idxy, bx, by, bo, semx, semy):
        wid = lax.axis_index("s") * _NC + lax.axis_index("c")
        base = wid * rows_per_tile

        def chunk(t, carry):
            off = base + t * C
            pltpu.sync_copy(xs_hbm.at[pl.ds(off, C)], idxx)
            pltpu.sync_copy(ys_hbm.at[pl.ds(off, C)], idxy)
            cx = pltpu.async_copy(pe_hbm.at[idxx], bx, semx)
            cy = pltpu.async_copy(pe_hbm.at[idxy], by, semy)
            cx.wait()
            cy.wait()

            def row(i, c2):
                for j in range(D // _L):
                    s = pl.ds(j * _L, _L)
                    bo[i, s] = bx[i, s] + by[i, s]
                return c2

            lax.fori_loop(0, C, row, 0)
            pltpu.sync_copy(bo, out_hbm.at[pl.ds(off, C)])
            return carry

        lax.fori_loop(0, nchunks, chunk, 0)

    return k


def kernel(coords, pe):
    B, T, _ = coords.shape
    N = B * T
    D = pe.shape[1]
    flat = coords.reshape(N, 2)
    xs = flat[:, 0]
    ys = flat[:, 1]
    out = _make_sc_kernel(N, D, 256)(xs, ys, pe)
    return out.reshape(B, T, D)


# speedup vs baseline: 7.5747x; 1.1014x over previous
import functools

import jax
import jax.numpy as jnp
from jax import lax
from jax.experimental import pallas as pl
from jax.experimental.pallas import tpu as pltpu
from jax.experimental.pallas import tpu_sc as plsc

_NC = 2
_NS = 16
_NW = _NC * _NS
_L = 16


@functools.lru_cache(maxsize=None)
def _make_sc_kernel(N, D, C):
    assert N % _NW == 0
    rows_per_tile = N // _NW
    assert rows_per_tile % C == 0
    nchunks = rows_per_tile // C
    assert nchunks % 2 == 0 and nchunks >= 4
    assert D % _L == 0 and (2 * C) % 8 == 0

    mesh = plsc.VectorSubcoreMesh(core_axis_name="c", subcore_axis_name="s")

    @functools.partial(
        pl.kernel,
        out_type=jax.ShapeDtypeStruct((N, D), jnp.float32),
        mesh=mesh,
        scratch_types=[
            pltpu.VMEM((2 * C,), jnp.int32),
            pltpu.VMEM((2 * C,), jnp.int32),
            pltpu.VMEM((2 * C, D), jnp.float32),
            pltpu.VMEM((2 * C, D), jnp.float32),
            pltpu.SemaphoreType.DMA,
            pltpu.SemaphoreType.DMA,
            pltpu.SemaphoreType.DMA,
            pltpu.SemaphoreType.DMA,
            pltpu.SemaphoreType.DMA,
            pltpu.SemaphoreType.DMA,
        ],
    )
    def k(idx_hbm, pe_hbm, out_hbm, ix0, ix1, br0, br1,
          si0, si1, sg0, sg1, so0, so1):
        wid = lax.axis_index("s") * _NC + lax.axis_index("c")
        base = wid * rows_per_tile
        ibufs, rbufs = (ix0, ix1), (br0, br1)
        isems, gsems, osems = (si0, si1), (sg0, sg1), (so0, so1)

        def fire_idx(t, p):
            src = idx_hbm.at[pl.ds(2 * (base + t * C), 2 * C)]
            pltpu.async_copy(src, ibufs[p], isems[p])

        def fire_gather(p):
            pltpu.async_copy(pe_hbm.at[ibufs[p]], rbufs[p], gsems[p])

        def fire_out(t, p):
            pltpu.async_copy(rbufs[p].at[pl.ds(0, C)],
                             out_hbm.at[pl.ds(base + t * C, C)], osems[p])

        def wait_idx(p):
            pltpu.make_async_copy(idx_hbm.at[pl.ds(0, 2 * C)],
                                  ibufs[p], isems[p]).wait()

        def wait_gather(p):
            pltpu.make_async_copy(pe_hbm.at[ibufs[p]],
                                  rbufs[p], gsems[p]).wait()

        def wait_out(p):
            pltpu.make_async_copy(rbufs[p].at[pl.ds(0, C)],
                                  out_hbm.at[pl.ds(base, C)], osems[p]).wait()

        pltpu.sync_copy(idx_hbm.at[pl.ds(2 * base, 2 * C)], ibufs[0])
        fire_gather(0)
        fire_idx(1, 1)

        def do_chunk(t, p):
            q = 1 - p
            wait_gather(p)

            @pl.when(t + 2 < nchunks)
            def _():
                fire_idx(t + 2, p)

            @pl.when(t + 1 < nchunks)
            def _():
                wait_idx(q)

                @pl.when(t >= 1)
                def _():
                    wait_out(q)

                fire_gather(q)

            def row(i, c2):
                for j in range(D // _L):
                    s = pl.ds(j * _L, _L)
                    plsc.addupdate(rbufs[p].at[i, s], rbufs[p][C + i, s])
                return c2

            lax.fori_loop(0, C, row, 0)
            fire_out(t, p)

        def two_chunks(kk, carry):
            do_chunk(2 * kk, 0)
            do_chunk(2 * kk + 1, 1)
            return carry

        lax.fori_loop(0, nchunks // 2, two_chunks, 0)
        wait_out(1)

    return k


def kernel(coords, pe):
    B, T, _ = coords.shape
    N = B * T
    D = pe.shape[1]
    C = 200
    idx_all = jnp.swapaxes(coords.reshape(N // C, C, 2), 1, 2).reshape(2 * N)
    out = _make_sc_kernel(N, D, C)(idx_all, pe)
    return out.reshape(B, T, D)


# speedup vs baseline: 14.7988x; 1.9537x over previous
import functools

import jax
import jax.numpy as jnp
from jax import lax
from jax.experimental import pallas as pl
from jax.experimental.pallas import tpu as pltpu
from jax.experimental.pallas import tpu_sc as plsc

_NC = 2
_NS = 16
_NW = _NC * _NS
_L = 16


@functools.lru_cache(maxsize=None)
def _make_sc_kernel(N, D, V, C):
    assert N % _NW == 0
    rows_per_tile = N // _NW
    assert rows_per_tile % C == 0
    nchunks = rows_per_tile // C
    assert nchunks % 2 == 0 and nchunks >= 4
    assert D % _L == 0 and (2 * C) % 8 == 0
    stage_tiles = next(nt for nt in range(_NS, 0, -1)
                       if V % nt == 0 and (V // nt) % 8 == 0)
    stage_rows = V // stage_tiles
    assert stage_rows <= 2 * C

    mesh = plsc.VectorSubcoreMesh(core_axis_name="c", subcore_axis_name="s")

    @functools.partial(
        pl.kernel,
        out_type=jax.ShapeDtypeStruct((N, D), jnp.float32),
        mesh=mesh,
        scratch_types=[
            pltpu.VMEM((2 * C,), jnp.int32),
            pltpu.VMEM((2 * C,), jnp.int32),
            pltpu.VMEM((2 * C, D), jnp.float32),
            pltpu.VMEM((2 * C, D), jnp.float32),
            pltpu.VMEM_SHARED((V, D), jnp.float32),
            pltpu.SemaphoreType.DMA,
            pltpu.SemaphoreType.DMA,
            pltpu.SemaphoreType.DMA,
            pltpu.SemaphoreType.DMA,
            pltpu.SemaphoreType.DMA,
            pltpu.SemaphoreType.DMA,
        ],
    )
    def k(idx_hbm, pe_hbm, out_hbm, ix0, ix1, br0, br1, pe_sh,
          si0, si1, sg0, sg1, so0, so1):
        sid = lax.axis_index("s")
        wid = sid * _NC + lax.axis_index("c")
        base = wid * rows_per_tile
        ibufs, rbufs = (ix0, ix1), (br0, br1)
        isems, gsems, osems = (si0, si1), (sg0, sg1), (so0, so1)

        srow = sid * stage_rows

        @pl.when(sid < stage_tiles)
        def _():
            pltpu.sync_copy(pe_hbm.at[pl.ds(srow, stage_rows)],
                            br0.at[pl.ds(0, stage_rows)])
            pltpu.sync_copy(br0.at[pl.ds(0, stage_rows)],
                            pe_sh.at[pl.ds(srow, stage_rows)])

        plsc.subcore_barrier()

        def fire_idx(t, p):
            src = idx_hbm.at[pl.ds(2 * (base + t * C), 2 * C)]
            pltpu.async_copy(src, ibufs[p], isems[p])

        def fire_gather(p):
            pltpu.async_copy(pe_sh.at[ibufs[p]], rbufs[p], gsems[p])

        def fire_out(t, p):
            pltpu.async_copy(rbufs[p].at[pl.ds(0, C)],
                             out_hbm.at[pl.ds(base + t * C, C)], osems[p])

        def wait_idx(p):
            pltpu.make_async_copy(idx_hbm.at[pl.ds(0, 2 * C)],
                                  ibufs[p], isems[p]).wait()

        def wait_gather(p):
            pltpu.make_async_copy(pe_sh.at[ibufs[p]],
                                  rbufs[p], gsems[p]).wait()

        def wait_out(p):
            pltpu.make_async_copy(rbufs[p].at[pl.ds(0, C)],
                                  out_hbm.at[pl.ds(base, C)], osems[p]).wait()

        pltpu.sync_copy(idx_hbm.at[pl.ds(2 * base, 2 * C)], ibufs[0])
        fire_gather(0)
        fire_idx(1, 1)

        def do_chunk(t, p):
            q = 1 - p
            wait_gather(p)

            @pl.when(t + 2 < nchunks)
            def _():
                fire_idx(t + 2, p)

            @pl.when(t + 1 < nchunks)
            def _():
                wait_idx(q)

                @pl.when(t >= 1)
                def _():
                    wait_out(q)

                fire_gather(q)

            def row(i, c2):
                for j in range(D // _L):
                    s = pl.ds(j * _L, _L)
                    plsc.addupdate(rbufs[p].at[i, s], rbufs[p][C + i, s])
                return c2

            lax.fori_loop(0, C, row, 0)
            fire_out(t, p)

        def two_chunks(kk, carry):
            do_chunk(2 * kk, 0)
            do_chunk(2 * kk + 1, 1)
            return carry

        lax.fori_loop(0, nchunks // 2, two_chunks, 0)
        wait_out(1)

    return k


def kernel(coords, pe):
    B, T, _ = coords.shape
    N = B * T
    D = pe.shape[1]
    C = 200
    idx_all = jnp.swapaxes(coords.reshape(N // C, C, 2), 1, 2).reshape(2 * N)
    out = _make_sc_kernel(N, D, pe.shape[0], C)(idx_all, pe)
    return out.reshape(B, T, D)
